# Initial kernel scaffold; baseline (speedup 1.0000x reference)
import functools

import jax
import jax.numpy as jnp
from jax import lax
from jax.experimental import pallas as pl
from jax.experimental.pallas import tpu as pltpu
from jax.experimental.pallas import tpu_sc as plsc

N = 10000
E = 160000
C = 128
EA = 16
SH = 25
H = 32

NC = 2
NS = 16
L = 16
NW = NC * NS
K = 128
NCH = E // K
CPW = -(-NCH // NW)
RPT = N // NS

_mesh = plsc.VectorSubcoreMesh(core_axis_name="c", subcore_axis_name="s")


def _mul_inplace(acc, other, rows):
    def row(r, carry):
        for cc in range(C // L):
            sl = pl.ds(cc * L, L)
            acc[r, sl] = acc[r, sl] * other[r, sl]
        return carry
    lax.fori_loop(0, rows, row, 0)


@functools.partial(
    pl.kernel,
    out_type=(
        jax.ShapeDtypeStruct((E, C), jnp.float32),
        jax.ShapeDtypeStruct((E, H), jnp.float32),
    ),
    mesh=_mesh,
    scratch_types=[
        pltpu.VMEM((K,), jnp.int32),
        pltpu.VMEM((K,), jnp.int32),
        pltpu.VMEM((K, C), jnp.float32),
        pltpu.VMEM((K, C), jnp.float32),
        pltpu.VMEM((K, H), jnp.float32),
        pltpu.SemaphoreType.DMA,
        pltpu.SemaphoreType.DMA,
        pltpu.SemaphoreType.DMA,
    ],
)
def _sc_gather(pre_hbm, an_hbm, dst_hbm, src_hbm, prod_hbm, t1_hbm,
               dstv, srcv, bufd, bufs, buft, semd, sems, semt):
    wid = lax.axis_index("s") * NC + lax.axis_index("c")

    def chunk(t, carry):
        cid = wid + NW * t

        @pl.when(cid < NCH)
        def _():
            base = cid * K
            pltpu.sync_copy(dst_hbm.at[pl.ds(base, K)], dstv)
            pltpu.sync_copy(src_hbm.at[pl.ds(base, K)], srcv)
            cd = pltpu.async_copy(pre_hbm.at[dstv], bufd, semd)
            cs = pltpu.async_copy(pre_hbm.at[srcv], bufs, sems)
            ct = pltpu.async_copy(an_hbm.at[dstv], buft, semt)
            cd.wait()
            cs.wait()
            _mul_inplace(bufd, bufs, K)
            pltpu.sync_copy(bufd, prod_hbm.at[pl.ds(base, K)])
            ct.wait()
            pltpu.sync_copy(buft, t1_hbm.at[pl.ds(base, K)])

        return carry

    lax.fori_loop(0, CPW, chunk, 0)


@functools.partial(
    pl.kernel,
    out_type=jax.ShapeDtypeStruct((NC * N, C), jnp.float32),
    mesh=_mesh,
    scratch_types=[
        pltpu.VMEM((K,), jnp.int32),
        pltpu.VMEM((K,), jnp.int32),
        pltpu.VMEM((K, C), jnp.float32),
        pltpu.VMEM((K, C), jnp.float32),
        pltpu.VMEM_SHARED((N, C), jnp.float32),
        pltpu.SemaphoreType.DMA,
        pltpu.SemaphoreType.DMA,
    ],
)
def _sc_scatter(g_hbm, xn_hbm, dst_hbm, src_hbm, out_hbm,
                dstv, srcv, bufx, bufg, agg, semx, semg):
    c = lax.axis_index("c")
    s = lax.axis_index("s")
    wid = s * NC + c

    def zrow(r, carry):
        for cc in range(C // L):
            bufx[r, pl.ds(cc * L, L)] = jnp.zeros((L,), jnp.float32)
        return carry
    lax.fori_loop(0, K, zrow, 0)
    for j in range(4):
        pltpu.sync_copy(bufx, agg.at[pl.ds(s * RPT + j * K, K)])
    pltpu.sync_copy(bufx.at[pl.ds(0, RPT - 4 * K)],
                    agg.at[pl.ds(s * RPT + 4 * K, RPT - 4 * K)])
    plsc.subcore_barrier()

    def chunk(t, carry):
        cid = wid + NW * t

        @pl.when(cid < NCH)
        def _():
            base = cid * K
            pltpu.sync_copy(dst_hbm.at[pl.ds(base, K)], dstv)
            pltpu.sync_copy(src_hbm.at[pl.ds(base, K)], srcv)
            cx = pltpu.async_copy(xn_hbm.at[srcv], bufx, semx)
            cg = pltpu.async_copy(g_hbm.at[pl.ds(base, K)], bufg, semg)
            cx.wait()
            cg.wait()
            _mul_inplace(bufx, bufg, K)
            pltpu.sync_copy(bufx, agg.at[dstv], add=True)

        return carry

    lax.fori_loop(0, CPW, chunk, 0)
    plsc.subcore_barrier()
    pltpu.sync_copy(agg.at[pl.ds(s * RPT, RPT)],
                    out_hbm.at[pl.ds(c * N + s * RPT, RPT)])


def _node_body(x_ref, W_pre_ref, b_pre_ref, WlA_ref, bl1_ref, Wg1_ref,
               bg1_ref, Wg2_ref, bg2_ref, W_node_ref, b_node_ref,
               pre_ref, an_ref, xn_ref):
    x = x_ref[...]
    pre = jnp.dot(x, W_pre_ref[...], preferred_element_type=jnp.float32) + b_pre_ref[...]
    pre_ref[...] = pre
    an_ref[...] = jnp.dot(pre, WlA_ref[...], preferred_element_type=jnp.float32) + bl1_ref[...]
    h1 = jnp.dot(x, Wg1_ref[...], preferred_element_type=jnp.float32) + bg1_ref[...]
    h = jnp.dot(jax.nn.silu(h1), Wg2_ref[...], preferred_element_type=jnp.float32) + bg2_ref[...]
    xn_ref[...] = jnp.dot(x * h, W_node_ref[...], preferred_element_type=jnp.float32) + b_node_ref[...]


_NB = 1000


def _node_call(x, W_pre, b_pre, WlA, bl1, Wg1, bg1, Wg2, bg2, W_node, b_node):
    full = lambda r, c_: pl.BlockSpec((r, c_), lambda i: (0, 0))
    blk = lambda c_: pl.BlockSpec((_NB, c_), lambda i: (i, 0))
    return pl.pallas_call(
        _node_body,
        grid=(N // _NB,),
        in_specs=[
            blk(C),
            full(C, C), full(1, C), full(C, H), full(1, H),
            full(C, C), full(1, C), full(C, C), full(1, C),
            full(C, C), full(1, C),
        ],
        out_specs=[blk(C), blk(H), blk(C)],
        out_shape=[
            jax.ShapeDtypeStruct((N, C), jnp.float32),
            jax.ShapeDtypeStruct((N, H), jnp.float32),
            jax.ShapeDtypeStruct((N, C), jnp.float32),
        ],
    )(x, W_pre, b_pre, WlA, bl1, Wg1, bg1, Wg2, bg2, W_node, b_node)


def _edge_body(prod_ref, t1_ref, ea_ref, sh_ref, WlB_ref, Wl2_ref, bl2_ref,
               W1_ref, b1_ref, W2_ref, b2_ref, Wsh_ref, g_ref):
    u = jax.nn.silu(jnp.dot(ea_ref[...], W1_ref[...], preferred_element_type=jnp.float32) + b1_ref[...])
    w_r = jnp.dot(u, W2_ref[...], preferred_element_type=jnp.float32) + b2_ref[...]
    v = jax.nn.silu(jnp.dot(prod_ref[...], WlB_ref[...], preferred_element_type=jnp.float32) + t1_ref[...])
    w_s = jnp.dot(v, Wl2_ref[...], preferred_element_type=jnp.float32) + bl2_ref[...]
    sh_p = jnp.dot(sh_ref[...], Wsh_ref[...], preferred_element_type=jnp.float32)
    g_ref[...] = w_r * w_s * sh_p


_EB = 2000


def _edge_call(prod, t1, edge_attr, edge_sh, WlB, Wl2, bl2, W1, b1, W2, b2, W_sh):
    full = lambda r, c_: pl.BlockSpec((r, c_), lambda i: (0, 0))
    blk = lambda c_: pl.BlockSpec((_EB, c_), lambda i: (i, 0))
    return pl.pallas_call(
        _edge_body,
        grid=(E // _EB,),
        in_specs=[
            blk(C), blk(H), blk(EA), blk(SH),
            full(C, H), full(H, C), full(1, C),
            full(EA, H), full(1, H), full(H, C), full(1, C),
            full(SH, C),
        ],
        out_specs=blk(C),
        out_shape=jax.ShapeDtypeStruct((E, C), jnp.float32),
    )(prod, t1, edge_attr, edge_sh, WlB, Wl2, bl2, W1, b1, W2, b2, W_sh)


def _out_body(p0_ref, p1_ref, xn_ref, W_out_ref, b_out_ref, o_ref):
    acc = p0_ref[...] + p1_ref[...] + xn_ref[...]
    o_ref[...] = jnp.dot(acc, W_out_ref[...], preferred_element_type=jnp.float32) + b_out_ref[...]


def _out_call(part, xn, W_out, b_out):
    full = lambda r, c_: pl.BlockSpec((r, c_), lambda i: (0, 0))
    return pl.pallas_call(
        _out_body,
        grid=(N // _NB,),
        in_specs=[
            pl.BlockSpec((_NB, C), lambda i: (i, 0)),
            pl.BlockSpec((_NB, C), lambda i: (i + N // _NB, 0)),
            pl.BlockSpec((_NB, C), lambda i: (i, 0)),
            full(C, C), full(1, C),
        ],
        out_specs=pl.BlockSpec((_NB, C), lambda i: (i, 0)),
        out_shape=jax.ShapeDtypeStruct((N, C), jnp.float32),
    )(part, part, xn, W_out, b_out)


def kernel(x, edge_index, edge_attr, edge_sh, W_pre, b_pre, Wg1, bg1, Wg2,
           bg2, W_node, b_node, W1, b1, W2, b2, Wl1, bl1, Wl2, bl2, W_sh,
           W_out, b_out):
    dst = edge_index[0]
    src = edge_index[1]
    WlA = Wl1[:C]
    WlB = Wl1[C:]

    pre_x, a_node, xn = _node_call(
        x, W_pre, b_pre.reshape(1, C), WlA, bl1.reshape(1, H),
        Wg1, bg1.reshape(1, C), Wg2, bg2.reshape(1, C),
        W_node, b_node.reshape(1, C))

    prod, t1 = _sc_gather(pre_x, a_node, dst, src)

    g = _edge_call(prod, t1, edge_attr, edge_sh, WlB, Wl2,
                   bl2.reshape(1, C), W1, b1.reshape(1, H), W2,
                   b2.reshape(1, C), W_sh)

    part = _sc_scatter(g, xn, dst, src)

    return _out_call(part, xn, W_out, b_out.reshape(1, C))


# speedup vs baseline: 2.9014x; 2.9014x over previous
import functools

import jax
import jax.numpy as jnp
from jax import lax
from jax.experimental import pallas as pl
from jax.experimental.pallas import tpu as pltpu
from jax.experimental.pallas import tpu_sc as plsc

N = 10000
E = 160000
C = 128
EA = 16
SH = 25
H = 32

NC = 2
NS = 16
L = 16
NW = NC * NS
K = 128
NCH = E // K
CPW = -(-NCH // NW)
RPT = 624

_mesh = plsc.VectorSubcoreMesh(core_axis_name="c", subcore_axis_name="s")


def _mul_inplace(acc, other, rows):
    def row(r, carry):
        for cc in range(C // L):
            sl = pl.ds(cc * L, L)
            acc[r, sl] = acc[r, sl] * other[r, sl]
        return carry
    lax.fori_loop(0, rows, row, 0)


@functools.partial(
    pl.kernel,
    out_type=(
        jax.ShapeDtypeStruct((E, C), jnp.float32),
        jax.ShapeDtypeStruct((E, C), jnp.float32),
    ),
    mesh=_mesh,
    scratch_types=[
        pltpu.VMEM((K,), jnp.int32),
        pltpu.VMEM((K,), jnp.int32),
        pltpu.VMEM((K, C), jnp.float32),
        pltpu.VMEM((K, C), jnp.float32),
        pltpu.SemaphoreType.DMA,
        pltpu.SemaphoreType.DMA,
    ],
)
def _sc_gather(pre_hbm, dst_hbm, src_hbm, pred_hbm, prod_hbm,
               dstv, srcv, bufd, bufs, semd, sems):
    wid = lax.axis_index("s") * NC + lax.axis_index("c")

    def chunk(t, carry):
        cid = wid + NW * t

        @pl.when(cid < NCH)
        def _():
            base = cid * K
            pltpu.sync_copy(dst_hbm.at[pl.ds(base, K)], dstv)
            pltpu.sync_copy(src_hbm.at[pl.ds(base, K)], srcv)
            cd = pltpu.async_copy(pre_hbm.at[dstv], bufd, semd)
            cs = pltpu.async_copy(pre_hbm.at[srcv], bufs, sems)
            cd.wait()
            pltpu.sync_copy(bufd, pred_hbm.at[pl.ds(base, K)])
            cs.wait()
            _mul_inplace(bufd, bufs, K)
            pltpu.sync_copy(bufd, prod_hbm.at[pl.ds(base, K)])

        return carry

    lax.fori_loop(0, CPW, chunk, 0)


@functools.partial(
    pl.kernel,
    out_type=jax.ShapeDtypeStruct((NC * N, C), jnp.float32),
    mesh=_mesh,
    scratch_types=[
        pltpu.VMEM((K,), jnp.int32),
        pltpu.VMEM((K,), jnp.int32),
        pltpu.VMEM((K, C), jnp.float32),
        pltpu.VMEM((K, C), jnp.float32),
        pltpu.VMEM_SHARED((N, C), jnp.float32),
        pltpu.SemaphoreType.DMA,
        pltpu.SemaphoreType.DMA,
    ],
)
def _sc_scatter(g_hbm, xn_hbm, dst_hbm, src_hbm, out_hbm,
                dstv, srcv, bufx, bufg, agg, semx, semg):
    c = lax.axis_index("c")
    s = lax.axis_index("s")
    wid = s * NC + c

    def zrow(r, carry):
        for cc in range(C // L):
            bufx[r, pl.ds(cc * L, L)] = jnp.zeros((L,), jnp.float32)
        return carry
    lax.fori_loop(0, K, zrow, 0)
    for j in range(4):
        pltpu.sync_copy(bufx, agg.at[pl.ds(s * RPT + j * K, K)])
    pltpu.sync_copy(bufx.at[pl.ds(0, RPT - 4 * K)],
                    agg.at[pl.ds(s * RPT + 4 * K, RPT - 4 * K)])

    @pl.when(s == NS - 1)
    def _():
        pltpu.sync_copy(bufx.at[pl.ds(0, N - NS * RPT)],
                        agg.at[pl.ds(NS * RPT, N - NS * RPT)])

    plsc.subcore_barrier()

    def chunk(t, carry):
        cid = wid + NW * t

        @pl.when(cid < NCH)
        def _():
            base = cid * K
            pltpu.sync_copy(dst_hbm.at[pl.ds(base, K)], dstv)
            pltpu.sync_copy(src_hbm.at[pl.ds(base, K)], srcv)
            cx = pltpu.async_copy(xn_hbm.at[srcv], bufx, semx)
            cg = pltpu.async_copy(g_hbm.at[pl.ds(base, K)], bufg, semg)
            cx.wait()
            cg.wait()
            _mul_inplace(bufx, bufg, K)
            pltpu.sync_copy(bufx, agg.at[dstv], add=True)

        return carry

    lax.fori_loop(0, CPW, chunk, 0)
    plsc.subcore_barrier()
    pltpu.sync_copy(agg.at[pl.ds(s * RPT, RPT)],
                    out_hbm.at[pl.ds(c * N + s * RPT, RPT)])

    @pl.when(s == NS - 1)
    def _():
        pltpu.sync_copy(agg.at[pl.ds(NS * RPT, N - NS * RPT)],
                        out_hbm.at[pl.ds(c * N + NS * RPT, N - NS * RPT)])


def _node_body(x_ref, W_pre_ref, b_pre_ref, Wg1_ref,
               bg1_ref, Wg2_ref, bg2_ref, W_node_ref, b_node_ref,
               pre_ref, xn_ref):
    x = x_ref[...]
    pre = jnp.dot(x, W_pre_ref[...], preferred_element_type=jnp.float32) + b_pre_ref[...]
    pre_ref[...] = pre
    h1 = jnp.dot(x, Wg1_ref[...], preferred_element_type=jnp.float32) + bg1_ref[...]
    h = jnp.dot(jax.nn.silu(h1), Wg2_ref[...], preferred_element_type=jnp.float32) + bg2_ref[...]
    xn_ref[...] = jnp.dot(x * h, W_node_ref[...], preferred_element_type=jnp.float32) + b_node_ref[...]


_NB = 1000


def _node_call(x, W_pre, b_pre, Wg1, bg1, Wg2, bg2, W_node, b_node):
    full = lambda r, c_: pl.BlockSpec((r, c_), lambda i: (0, 0))
    blk = lambda c_: pl.BlockSpec((_NB, c_), lambda i: (i, 0))
    return pl.pallas_call(
        _node_body,
        grid=(N // _NB,),
        in_specs=[
            blk(C),
            full(C, C), full(1, C),
            full(C, C), full(1, C), full(C, C), full(1, C),
            full(C, C), full(1, C),
        ],
        out_specs=[blk(C), blk(C)],
        out_shape=[
            jax.ShapeDtypeStruct((N, C), jnp.float32),
            jax.ShapeDtypeStruct((N, C), jnp.float32),
        ],
    )(x, W_pre, b_pre, Wg1, bg1, Wg2, bg2, W_node, b_node)


def _edge_body(pred_ref, prod_ref, ea_ref, sh_ref, WlA_ref, WlB_ref, bl1_ref,
               Wl2_ref, bl2_ref, W1_ref, b1_ref, W2_ref, b2_ref, Wsh_ref,
               g_ref):
    u = jax.nn.silu(jnp.dot(ea_ref[...], W1_ref[...], preferred_element_type=jnp.float32) + b1_ref[...])
    w_r = jnp.dot(u, W2_ref[...], preferred_element_type=jnp.float32) + b2_ref[...]
    t = (jnp.dot(pred_ref[...], WlA_ref[...], preferred_element_type=jnp.float32)
         + jnp.dot(prod_ref[...], WlB_ref[...], preferred_element_type=jnp.float32)
         + bl1_ref[...])
    w_s = jnp.dot(jax.nn.silu(t), Wl2_ref[...], preferred_element_type=jnp.float32) + bl2_ref[...]
    sh_p = jnp.dot(sh_ref[...], Wsh_ref[...], preferred_element_type=jnp.float32)
    g_ref[...] = w_r * w_s * sh_p


_EB = 2000


def _edge_call(pre_d, prod, edge_attr, edge_sh, WlA, WlB, bl1, Wl2, bl2,
               W1, b1, W2, b2, W_sh):
    full = lambda r, c_: pl.BlockSpec((r, c_), lambda i: (0, 0))
    blk = lambda c_: pl.BlockSpec((_EB, c_), lambda i: (i, 0))
    return pl.pallas_call(
        _edge_body,
        grid=(E // _EB,),
        in_specs=[
            blk(C), blk(C), blk(EA), blk(SH),
            full(C, H), full(C, H), full(1, H),
            full(H, C), full(1, C),
            full(EA, H), full(1, H), full(H, C), full(1, C),
            full(SH, C),
        ],
        out_specs=blk(C),
        out_shape=jax.ShapeDtypeStruct((E, C), jnp.float32),
    )(pre_d, prod, edge_attr, edge_sh, WlA, WlB, bl1, Wl2, bl2,
      W1, b1, W2, b2, W_sh)


def _out_body(p0_ref, p1_ref, xn_ref, W_out_ref, b_out_ref, o_ref):
    acc = p0_ref[...] + p1_ref[...] + xn_ref[...]
    o_ref[...] = jnp.dot(acc, W_out_ref[...], preferred_element_type=jnp.float32) + b_out_ref[...]


def _out_call(part, xn, W_out, b_out):
    full = lambda r, c_: pl.BlockSpec((r, c_), lambda i: (0, 0))
    return pl.pallas_call(
        _out_body,
        grid=(N // _NB,),
        in_specs=[
            pl.BlockSpec((_NB, C), lambda i: (i, 0)),
            pl.BlockSpec((_NB, C), lambda i: (i + N // _NB, 0)),
            pl.BlockSpec((_NB, C), lambda i: (i, 0)),
            full(C, C), full(1, C),
        ],
        out_specs=pl.BlockSpec((_NB, C), lambda i: (i, 0)),
        out_shape=jax.ShapeDtypeStruct((N, C), jnp.float32),
    )(part, part, xn, W_out, b_out)


def kernel(x, edge_index, edge_attr, edge_sh, W_pre, b_pre, Wg1, bg1, Wg2,
           bg2, W_node, b_node, W1, b1, W2, b2, Wl1, bl1, Wl2, bl2, W_sh,
           W_out, b_out):
    dst = edge_index[0]
    src = edge_index[1]
    WlA = Wl1[:C]
    WlB = Wl1[C:]

    pre_x, xn = _node_call(
        x, W_pre, b_pre.reshape(1, C),
        Wg1, bg1.reshape(1, C), Wg2, bg2.reshape(1, C),
        W_node, b_node.reshape(1, C))

    pre_d, prod = _sc_gather(pre_x, dst, src)

    g = _edge_call(pre_d, prod, edge_attr, edge_sh, WlA, WlB,
                   bl1.reshape(1, H), Wl2, bl2.reshape(1, C),
                   W1, b1.reshape(1, H), W2, b2.reshape(1, C), W_sh)

    part = _sc_scatter(g, xn, dst, src)

    return _out_call(part, xn, W_out, b_out.reshape(1, C))
